# baseline (speedup 1.0000x reference)
import functools

import jax
import jax.numpy as jnp
from jax import lax
from jax.experimental import pallas as pl
from jax.experimental.pallas import tpu as pltpu
from jax.experimental.pallas import tpu_sc as plsc

N = 10000
NP = 10240
D = 128
DO = 18
E = 320000
NC = 2
NS = 16
NW = NC * NS
EPW = E // NW
RPT = NP // NS

C = 32
CHUNKS = EPW // C
NB = 7
GRP = CHUNKS // NB
TAIL = CHUNKS - GRP * NB
TAILE = EPW - CHUNKS * C

CD = 128
DCHUNKS = EPW // CD
DTAILE = EPW - DCHUNKS * CD
ND = 6
DGRP = DCHUNKS // ND
DTAIL = DCHUNKS - DGRP * ND


def _sc_deg_body(dst_hbm, out_a, out_b, deg_sp, z_v, ones_v, didxt, *bufs):
    didx = bufs[0:ND]
    sem_i = bufs[ND:2 * ND]
    sem_s = bufs[2 * ND:3 * ND]
    cid = lax.axis_index("c")
    sid = lax.axis_index("s")
    gw = cid * NS + sid
    ebase = gw * EPW

    def zfill(i, _):
        z_v[pl.ds(i * 16, 16)] = jnp.zeros((16,), jnp.float32)
        return 0
    lax.fori_loop(0, RPT // 16, zfill, 0)

    def ofill(i, _):
        ones_v[pl.ds(i * 16, 16)] = jnp.ones((16,), jnp.float32)
        return 0
    lax.fori_loop(0, CD // 16, ofill, 0)

    pltpu.sync_copy(z_v, deg_sp.at[pl.ds(sid * RPT, RPT)])
    plsc.subcore_barrier()

    def drain_scatter(b):
        pltpu.make_async_copy(ones_v, deg_sp.at[didx[b]], sem_s[b]).wait()

    def run_chunks(base, nb, first):
        idescs = []
        for b in range(nb):
            if not first:
                drain_scatter(b)
            idescs.append(
                pltpu.async_copy(dst_hbm.at[pl.ds(base + b * CD, CD)],
                                 didx[b], sem_i[b]))
        for b in range(nb):
            idescs[b].wait()
            pltpu.async_copy(ones_v, deg_sp.at[didx[b]], sem_s[b],
                             add=True)

    def group(g, _):
        @pl.when(g == 0)
        def _():
            run_chunks(ebase, ND, True)

        @pl.when(g > 0)
        def _():
            run_chunks(ebase + g * ND * CD, ND, False)
        return 0
    lax.fori_loop(0, DGRP, group, 0)
    for b in range(ND):
        drain_scatter(b)
    if DTAIL:
        run_chunks(ebase + DGRP * ND * CD, DTAIL, True)
        for b in range(DTAIL):
            drain_scatter(b)
    if DTAILE:
        pltpu.sync_copy(dst_hbm.at[pl.ds(ebase + DCHUNKS * CD, DTAILE)],
                        didxt)
        pltpu.sync_copy(ones_v.at[pl.ds(0, DTAILE)], deg_sp.at[didxt],
                        add=True)

    plsc.subcore_barrier()
    pltpu.sync_copy(deg_sp.at[pl.ds(sid * RPT, RPT)], z_v)

    @pl.when(cid == 0)
    def _():
        pltpu.sync_copy(z_v, out_a.at[pl.ds(sid * RPT, RPT)])

    @pl.when(cid == 1)
    def _():
        pltpu.sync_copy(z_v, out_b.at[pl.ds(sid * RPT, RPT)])


@functools.cache
def _sc_deg():
    mesh = plsc.VectorSubcoreMesh(
        core_axis_name="c", subcore_axis_name="s",
        num_cores=NC, num_subcores=NS)
    return pl.kernel(
        _sc_deg_body,
        out_type=(
            jax.ShapeDtypeStruct((NP,), jnp.float32),
            jax.ShapeDtypeStruct((NP,), jnp.float32),
        ),
        mesh=mesh,
        scratch_types=(
            [pltpu.VMEM_SHARED((NP,), jnp.float32),
             pltpu.VMEM((RPT,), jnp.float32),
             pltpu.VMEM((CD,), jnp.float32),
             pltpu.VMEM((max(DTAILE, 8),), jnp.int32)]
            + [pltpu.VMEM((CD,), jnp.int32)] * ND
            + [pltpu.SemaphoreType.DMA] * (2 * ND)
        ),
    )


def _sc_agg_body(hs_hbm, src_hbm, dst_hbm, zer_hbm, out_hbm, acc_sp,
                 srcall, didxt, *bufs):
    didx = bufs[0:NB]
    rows = bufs[NB:2 * NB]
    sem_i = bufs[2 * NB:3 * NB]
    sem_r = bufs[3 * NB:4 * NB]
    sem_s = bufs[4 * NB:5 * NB]
    cid = lax.axis_index("c")
    sid = lax.axis_index("s")
    gw = cid * NS + sid
    ebase = gw * EPW

    pltpu.sync_copy(src_hbm.at[pl.ds(ebase, EPW)], srcall)

    pltpu.sync_copy(zer_hbm.at[pl.ds(sid * RPT, RPT)],
                    acc_sp.at[pl.ds(sid * RPT, RPT)])
    plsc.subcore_barrier()

    def drain_scatter(b):
        pltpu.make_async_copy(rows[b], acc_sp.at[didx[b]], sem_s[b]).wait()

    def run_chunks(off, base, nb, first):
        gdescs = []
        idescs = []
        for b in range(nb):
            if not first:
                drain_scatter(b)
            gdescs.append(
                pltpu.async_copy(
                    hs_hbm.at[srcall.at[pl.ds(off + b * C, C)]],
                    rows[b], sem_r[b]))
            idescs.append(
                pltpu.async_copy(dst_hbm.at[pl.ds(base + b * C, C)],
                                 didx[b], sem_i[b]))
        for b in range(nb):
            idescs[b].wait()
            gdescs[b].wait()
            pltpu.async_copy(rows[b], acc_sp.at[didx[b]], sem_s[b],
                             add=True)

    def group(g, _):
        @pl.when(g == 0)
        def _():
            run_chunks(0, ebase, NB, True)

        @pl.when(g > 0)
        def _():
            run_chunks(g * NB * C, ebase + g * NB * C, NB, False)
        return 0
    lax.fori_loop(0, GRP, group, 0)
    for b in range(NB):
        drain_scatter(b)
    toff = GRP * NB * C
    for t in range(TAIL):
        pltpu.sync_copy(dst_hbm.at[pl.ds(ebase + toff + t * C, C)], didx[t])
        pltpu.async_copy(hs_hbm.at[srcall.at[pl.ds(toff + t * C, C)]],
                         rows[t], sem_r[t]).wait()
        pltpu.sync_copy(rows[t], acc_sp.at[didx[t]], add=True)
    if TAILE:
        eoff = CHUNKS * C
        pltpu.sync_copy(dst_hbm.at[pl.ds(ebase + eoff, TAILE)], didxt)
        pltpu.async_copy(hs_hbm.at[srcall.at[pl.ds(eoff, TAILE)]],
                         rows[0].at[pl.ds(0, TAILE)], sem_r[0]).wait()
        pltpu.sync_copy(rows[0].at[pl.ds(0, TAILE)], acc_sp.at[didxt],
                        add=True)

    plsc.subcore_barrier()
    pltpu.sync_copy(acc_sp.at[pl.ds(sid * RPT, RPT)],
                    out_hbm.at[cid, pl.ds(sid * RPT, RPT)])


@functools.cache
def _sc_agg():
    mesh = plsc.VectorSubcoreMesh(
        core_axis_name="c", subcore_axis_name="s",
        num_cores=NC, num_subcores=NS)
    return pl.kernel(
        _sc_agg_body,
        out_type=jax.ShapeDtypeStruct((NC, NP, D), jnp.float32),
        mesh=mesh,
        scratch_types=(
            [pltpu.VMEM_SHARED((NP, D), jnp.float32),
             pltpu.VMEM((EPW,), jnp.int32),
             pltpu.VMEM((max(TAILE, 8),), jnp.int32)]
            + [pltpu.VMEM((C,), jnp.int32)] * NB
            + [pltpu.VMEM((C, D), jnp.float32)] * NB
            + [pltpu.SemaphoreType.DMA] * (3 * NB)
        ),
    )


R = 1024
G = NP // R


def _tc_emb_body(x_ref, we_ref, be_ref, h0_ref):
    h0_ref[...] = jnp.dot(x_ref[...], we_ref[...],
                          preferred_element_type=jnp.float32) + be_ref[...]


_tc_emb = pl.pallas_call(
    _tc_emb_body,
    grid=(G,),
    in_specs=[
        pl.BlockSpec((R, D), lambda i: (i, 0)),
        pl.BlockSpec((D, D), lambda i: (0, 0)),
        pl.BlockSpec((1, D), lambda i: (0, 0)),
    ],
    out_specs=pl.BlockSpec((R, D), lambda i: (i, 0)),
    out_shape=jax.ShapeDtypeStruct((NP, D), jnp.float32),
)


def _tc_pre_body(h0_ref, w1_ref, da_ref, db_ref, hs_ref, dinv_ref):
    dinv = lax.rsqrt(da_ref[...] + db_ref[...] + 1.0)
    hs_ref[...] = jnp.dot(h0_ref[...], w1_ref[...],
                          preferred_element_type=jnp.float32) * dinv
    dinv_ref[...] = dinv


_tc_pre = pl.pallas_call(
    _tc_pre_body,
    grid=(G,),
    in_specs=[
        pl.BlockSpec((R, D), lambda i: (i, 0)),
        pl.BlockSpec((D, D), lambda i: (0, 0)),
        pl.BlockSpec((R, 1), lambda i: (i, 0)),
        pl.BlockSpec((R, 1), lambda i: (i, 0)),
    ],
    out_specs=[
        pl.BlockSpec((R, D), lambda i: (i, 0)),
        pl.BlockSpec((R, 1), lambda i: (i, 0)),
    ],
    out_shape=[
        jax.ShapeDtypeStruct((NP, D), jnp.float32),
        jax.ShapeDtypeStruct((NP, 1), jnp.float32),
    ],
)


def _tc_mid_body(agg_ref, hs_ref, dinv_ref, b_ref, w_ref, out_ref):
    a = agg_ref[...]
    dinv = dinv_ref[...]
    h = (a[0] + a[1] + hs_ref[...]) * dinv + b_ref[...]
    out_ref[...] = jnp.dot(h, w_ref[...],
                           preferred_element_type=jnp.float32) * dinv


_tc_mid = pl.pallas_call(
    _tc_mid_body,
    grid=(G,),
    in_specs=[
        pl.BlockSpec((NC, R, D), lambda i: (0, i, 0)),
        pl.BlockSpec((R, D), lambda i: (i, 0)),
        pl.BlockSpec((R, 1), lambda i: (i, 0)),
        pl.BlockSpec((1, D), lambda i: (0, 0)),
        pl.BlockSpec((D, D), lambda i: (0, 0)),
    ],
    out_specs=pl.BlockSpec((R, D), lambda i: (i, 0)),
    out_shape=jax.ShapeDtypeStruct((NP, D), jnp.float32),
)


def _tc_dec_body(agg_ref, hs_ref, dinv_ref, b_ref, wd_ref, bd_ref, out_ref):
    a = agg_ref[...]
    h = (a[0] + a[1] + hs_ref[...]) * dinv_ref[...] + b_ref[...]
    out_ref[...] = jnp.dot(h, wd_ref[...],
                           preferred_element_type=jnp.float32) + bd_ref[...]


_tc_dec = pl.pallas_call(
    _tc_dec_body,
    grid=(G,),
    in_specs=[
        pl.BlockSpec((NC, R, D), lambda i: (0, i, 0)),
        pl.BlockSpec((R, D), lambda i: (i, 0)),
        pl.BlockSpec((R, 1), lambda i: (i, 0)),
        pl.BlockSpec((1, D), lambda i: (0, 0)),
        pl.BlockSpec((D, DO), lambda i: (0, 0)),
        pl.BlockSpec((1, DO), lambda i: (0, 0)),
    ],
    out_specs=pl.BlockSpec((R, DO), lambda i: (i, 0)),
    out_shape=jax.ShapeDtypeStruct((NP, DO), jnp.float32),
)


def kernel(x, edge_index, w_emb, b_emb, w1, b1, w2, b2, w3, b3, w_dec, b_dec):
    xp = jnp.pad(x, ((0, NP - N), (0, 0)))
    src = edge_index[0]
    dst = edge_index[1]
    zer = jnp.zeros((NP, D), jnp.float32)

    deg_a, deg_b = _sc_deg()(dst)
    h0 = _tc_emb(xp, w_emb, b_emb.reshape(1, D))
    da = deg_a.reshape(NP, 1)
    db = deg_b.reshape(NP, 1)

    hs, dinv = _tc_pre(h0, w1, da, db)
    agg = _sc_agg()(hs, src, dst, zer)
    hs = _tc_mid(agg, hs, dinv, b1.reshape(1, D), w2)
    agg = _sc_agg()(hs, src, dst, zer)
    hs = _tc_mid(agg, hs, dinv, b2.reshape(1, D), w3)
    agg = _sc_agg()(hs, src, dst, zer)
    out = _tc_dec(agg, hs, dinv, b3.reshape(1, D), w_dec, b_dec.reshape(1, DO))
    return out[:N]


# speedup vs baseline: 1.0457x; 1.0457x over previous
import functools

import jax
import jax.numpy as jnp
from jax import lax
from jax.experimental import pallas as pl
from jax.experimental.pallas import tpu as pltpu
from jax.experimental.pallas import tpu_sc as plsc

N = 10000
NP = 10240
D = 128
DO = 18
E = 320000
NC = 2
NS = 16
NW = NC * NS
EPW = E // NW
RPT = NP // NS

C = 40
CHUNKS = EPW // C
NB = 6
GRP = CHUNKS // NB
TAIL = CHUNKS - GRP * NB
TAILE = EPW - CHUNKS * C

CD = 128
DCHUNKS = EPW // CD
DTAILE = EPW - DCHUNKS * CD
ND = 6
DGRP = DCHUNKS // ND
DTAIL = DCHUNKS - DGRP * ND


def _sc_deg_body(dst_hbm, out_a, out_b, deg_sp, z_v, ones_v, didxt, *bufs):
    didx = bufs[0:ND]
    sem_i = bufs[ND:2 * ND]
    sem_s = bufs[2 * ND:3 * ND]
    cid = lax.axis_index("c")
    sid = lax.axis_index("s")
    gw = cid * NS + sid
    ebase = gw * EPW

    def zfill(i, _):
        z_v[pl.ds(i * 16, 16)] = jnp.zeros((16,), jnp.float32)
        return 0
    lax.fori_loop(0, RPT // 16, zfill, 0)

    def ofill(i, _):
        ones_v[pl.ds(i * 16, 16)] = jnp.ones((16,), jnp.float32)
        return 0
    lax.fori_loop(0, CD // 16, ofill, 0)

    pltpu.sync_copy(z_v, deg_sp.at[pl.ds(sid * RPT, RPT)])
    plsc.subcore_barrier()

    def drain_scatter(b):
        pltpu.make_async_copy(ones_v, deg_sp.at[didx[b]], sem_s[b]).wait()

    def run_chunks(base, nb, first):
        idescs = []
        for b in range(nb):
            if not first:
                drain_scatter(b)
            idescs.append(
                pltpu.async_copy(dst_hbm.at[pl.ds(base + b * CD, CD)],
                                 didx[b], sem_i[b]))
        for b in range(nb):
            idescs[b].wait()
            pltpu.async_copy(ones_v, deg_sp.at[didx[b]], sem_s[b],
                             add=True)

    def group(g, _):
        @pl.when(g == 0)
        def _():
            run_chunks(ebase, ND, True)

        @pl.when(g > 0)
        def _():
            run_chunks(ebase + g * ND * CD, ND, False)
        return 0
    lax.fori_loop(0, DGRP, group, 0)
    for b in range(ND):
        drain_scatter(b)
    if DTAIL:
        run_chunks(ebase + DGRP * ND * CD, DTAIL, True)
        for b in range(DTAIL):
            drain_scatter(b)
    if DTAILE:
        pltpu.sync_copy(dst_hbm.at[pl.ds(ebase + DCHUNKS * CD, DTAILE)],
                        didxt)
        pltpu.sync_copy(ones_v.at[pl.ds(0, DTAILE)], deg_sp.at[didxt],
                        add=True)

    plsc.subcore_barrier()
    pltpu.sync_copy(deg_sp.at[pl.ds(sid * RPT, RPT)], z_v)

    @pl.when(cid == 0)
    def _():
        pltpu.sync_copy(z_v, out_a.at[pl.ds(sid * RPT, RPT)])

    @pl.when(cid == 1)
    def _():
        pltpu.sync_copy(z_v, out_b.at[pl.ds(sid * RPT, RPT)])


@functools.cache
def _sc_deg():
    mesh = plsc.VectorSubcoreMesh(
        core_axis_name="c", subcore_axis_name="s",
        num_cores=NC, num_subcores=NS)
    return pl.kernel(
        _sc_deg_body,
        out_type=(
            jax.ShapeDtypeStruct((NP,), jnp.float32),
            jax.ShapeDtypeStruct((NP,), jnp.float32),
        ),
        mesh=mesh,
        scratch_types=(
            [pltpu.VMEM_SHARED((NP,), jnp.float32),
             pltpu.VMEM((RPT,), jnp.float32),
             pltpu.VMEM((CD,), jnp.float32),
             pltpu.VMEM((max(DTAILE, 8),), jnp.int32)]
            + [pltpu.VMEM((CD,), jnp.int32)] * ND
            + [pltpu.SemaphoreType.DMA] * (2 * ND)
        ),
    )


def _sc_agg_body(hs_hbm, src_hbm, dst_hbm, zer_hbm, out_hbm, acc_sp,
                 srcall, didxt, *bufs):
    didx = bufs[0:NB]
    rows = bufs[NB:2 * NB]
    sem_i = bufs[2 * NB:3 * NB]
    sem_r = bufs[3 * NB:4 * NB]
    sem_s = bufs[4 * NB:5 * NB]
    cid = lax.axis_index("c")
    sid = lax.axis_index("s")
    gw = cid * NS + sid
    ebase = gw * EPW

    pltpu.sync_copy(src_hbm.at[pl.ds(ebase, EPW)], srcall)

    pltpu.sync_copy(zer_hbm.at[pl.ds(sid * RPT, RPT)],
                    acc_sp.at[pl.ds(sid * RPT, RPT)])
    plsc.subcore_barrier()

    def drain_scatter(b):
        pltpu.make_async_copy(rows[b], acc_sp.at[didx[b]], sem_s[b]).wait()

    def run_chunks(off, base, nb, first):
        gdescs = []
        idescs = []
        for b in range(nb):
            if not first:
                drain_scatter(b)
            gdescs.append(
                pltpu.async_copy(
                    hs_hbm.at[srcall.at[pl.ds(off + b * C, C)]],
                    rows[b], sem_r[b]))
            idescs.append(
                pltpu.async_copy(dst_hbm.at[pl.ds(base + b * C, C)],
                                 didx[b], sem_i[b]))
        for b in range(nb):
            idescs[b].wait()
            gdescs[b].wait()
            pltpu.async_copy(rows[b], acc_sp.at[didx[b]], sem_s[b],
                             add=True)

    def group(g, _):
        @pl.when(g == 0)
        def _():
            run_chunks(0, ebase, NB, True)

        @pl.when(g > 0)
        def _():
            run_chunks(g * NB * C, ebase + g * NB * C, NB, False)
        return 0
    lax.fori_loop(0, GRP, group, 0)
    for b in range(NB):
        drain_scatter(b)
    toff = GRP * NB * C
    for t in range(TAIL):
        pltpu.sync_copy(dst_hbm.at[pl.ds(ebase + toff + t * C, C)], didx[t])
        pltpu.async_copy(hs_hbm.at[srcall.at[pl.ds(toff + t * C, C)]],
                         rows[t], sem_r[t]).wait()
        pltpu.sync_copy(rows[t], acc_sp.at[didx[t]], add=True)
    if TAILE:
        eoff = CHUNKS * C
        pltpu.sync_copy(dst_hbm.at[pl.ds(ebase + eoff, TAILE)], didxt)
        pltpu.async_copy(hs_hbm.at[srcall.at[pl.ds(eoff, TAILE)]],
                         rows[0].at[pl.ds(0, TAILE)], sem_r[0]).wait()
        pltpu.sync_copy(rows[0].at[pl.ds(0, TAILE)], acc_sp.at[didxt],
                        add=True)

    plsc.subcore_barrier()
    pltpu.sync_copy(acc_sp.at[pl.ds(sid * RPT, RPT)],
                    out_hbm.at[cid, pl.ds(sid * RPT, RPT)])


@functools.cache
def _sc_agg():
    mesh = plsc.VectorSubcoreMesh(
        core_axis_name="c", subcore_axis_name="s",
        num_cores=NC, num_subcores=NS)
    return pl.kernel(
        _sc_agg_body,
        out_type=jax.ShapeDtypeStruct((NC, NP, D), jnp.float32),
        mesh=mesh,
        scratch_types=(
            [pltpu.VMEM_SHARED((NP, D), jnp.float32),
             pltpu.VMEM((EPW,), jnp.int32),
             pltpu.VMEM((max(TAILE, 8),), jnp.int32)]
            + [pltpu.VMEM((C,), jnp.int32)] * NB
            + [pltpu.VMEM((C, D), jnp.float32)] * NB
            + [pltpu.SemaphoreType.DMA] * (3 * NB)
        ),
    )


R = 2048
G = NP // R


def _tc_emb_body(x_ref, we_ref, be_ref, h0_ref):
    h0_ref[...] = jnp.dot(x_ref[...], we_ref[...],
                          preferred_element_type=jnp.float32) + be_ref[...]


_tc_emb = pl.pallas_call(
    _tc_emb_body,
    grid=(G,),
    in_specs=[
        pl.BlockSpec((R, D), lambda i: (i, 0)),
        pl.BlockSpec((D, D), lambda i: (0, 0)),
        pl.BlockSpec((1, D), lambda i: (0, 0)),
    ],
    out_specs=pl.BlockSpec((R, D), lambda i: (i, 0)),
    out_shape=jax.ShapeDtypeStruct((NP, D), jnp.float32),
)


def _tc_pre_body(h0_ref, w1_ref, da_ref, db_ref, hs_ref, dinv_ref):
    dinv = lax.rsqrt(da_ref[...] + db_ref[...] + 1.0)
    hs_ref[...] = jnp.dot(h0_ref[...], w1_ref[...],
                          preferred_element_type=jnp.float32) * dinv
    dinv_ref[...] = dinv


_tc_pre = pl.pallas_call(
    _tc_pre_body,
    grid=(G,),
    in_specs=[
        pl.BlockSpec((R, D), lambda i: (i, 0)),
        pl.BlockSpec((D, D), lambda i: (0, 0)),
        pl.BlockSpec((R, 1), lambda i: (i, 0)),
        pl.BlockSpec((R, 1), lambda i: (i, 0)),
    ],
    out_specs=[
        pl.BlockSpec((R, D), lambda i: (i, 0)),
        pl.BlockSpec((R, 1), lambda i: (i, 0)),
    ],
    out_shape=[
        jax.ShapeDtypeStruct((NP, D), jnp.float32),
        jax.ShapeDtypeStruct((NP, 1), jnp.float32),
    ],
)


def _tc_mid_body(agg_ref, hs_ref, dinv_ref, b_ref, w_ref, out_ref):
    a = agg_ref[...]
    dinv = dinv_ref[...]
    h = (a[0] + a[1] + hs_ref[...]) * dinv + b_ref[...]
    out_ref[...] = jnp.dot(h, w_ref[...],
                           preferred_element_type=jnp.float32) * dinv


_tc_mid = pl.pallas_call(
    _tc_mid_body,
    grid=(G,),
    in_specs=[
        pl.BlockSpec((NC, R, D), lambda i: (0, i, 0)),
        pl.BlockSpec((R, D), lambda i: (i, 0)),
        pl.BlockSpec((R, 1), lambda i: (i, 0)),
        pl.BlockSpec((1, D), lambda i: (0, 0)),
        pl.BlockSpec((D, D), lambda i: (0, 0)),
    ],
    out_specs=pl.BlockSpec((R, D), lambda i: (i, 0)),
    out_shape=jax.ShapeDtypeStruct((NP, D), jnp.float32),
)


def _tc_dec_body(agg_ref, hs_ref, dinv_ref, b_ref, wd_ref, bd_ref, out_ref):
    a = agg_ref[...]
    h = (a[0] + a[1] + hs_ref[...]) * dinv_ref[...] + b_ref[...]
    out_ref[...] = jnp.dot(h, wd_ref[...],
                           preferred_element_type=jnp.float32) + bd_ref[...]


_tc_dec = pl.pallas_call(
    _tc_dec_body,
    grid=(G,),
    in_specs=[
        pl.BlockSpec((NC, R, D), lambda i: (0, i, 0)),
        pl.BlockSpec((R, D), lambda i: (i, 0)),
        pl.BlockSpec((R, 1), lambda i: (i, 0)),
        pl.BlockSpec((1, D), lambda i: (0, 0)),
        pl.BlockSpec((D, DO), lambda i: (0, 0)),
        pl.BlockSpec((1, DO), lambda i: (0, 0)),
    ],
    out_specs=pl.BlockSpec((R, DO), lambda i: (i, 0)),
    out_shape=jax.ShapeDtypeStruct((NP, DO), jnp.float32),
)


def kernel(x, edge_index, w_emb, b_emb, w1, b1, w2, b2, w3, b3, w_dec, b_dec):
    xp = jnp.pad(x, ((0, NP - N), (0, 0)))
    src = edge_index[0]
    dst = edge_index[1]
    zer = jnp.zeros((NP, D), jnp.float32)

    deg_a, deg_b = _sc_deg()(dst)
    h0 = _tc_emb(xp, w_emb, b_emb.reshape(1, D))
    da = deg_a.reshape(NP, 1)
    db = deg_b.reshape(NP, 1)

    hs, dinv = _tc_pre(h0, w1, da, db)
    agg = _sc_agg()(hs, src, dst, zer)
    hs = _tc_mid(agg, hs, dinv, b1.reshape(1, D), w2)
    agg = _sc_agg()(hs, src, dst, zer)
    hs = _tc_mid(agg, hs, dinv, b2.reshape(1, D), w3)
    agg = _sc_agg()(hs, src, dst, zer)
    out = _tc_dec(agg, hs, dinv, b3.reshape(1, D), w_dec, b_dec.reshape(1, DO))
    return out[:N]


# speedup vs baseline: 1.0498x; 1.0040x over previous
import functools

import jax
import jax.numpy as jnp
from jax import lax
from jax.experimental import pallas as pl
from jax.experimental.pallas import tpu as pltpu
from jax.experimental.pallas import tpu_sc as plsc

N = 10000
NP = 10240
D = 128
DO = 18
E = 320000
NC = 2
NS = 16
NW = NC * NS
EPW = E // NW
RPT = NP // NS

C = 40
CHUNKS = EPW // C
NB = 6
GRP = CHUNKS // NB
TAIL = CHUNKS - GRP * NB
TAILE = EPW - CHUNKS * C

CD = 128
DCHUNKS = EPW // CD
DTAILE = EPW - DCHUNKS * CD
ND = 8
DGRP = DCHUNKS // ND
DTAIL = DCHUNKS - DGRP * ND


def _sc_deg_body(dst_hbm, out_a, out_b, deg_sp, z_v, ones_v, didxt, *bufs):
    didx = bufs[0:ND]
    sem_i = bufs[ND:2 * ND]
    sem_s = bufs[2 * ND:3 * ND]
    cid = lax.axis_index("c")
    sid = lax.axis_index("s")
    gw = cid * NS + sid
    ebase = gw * EPW

    def zfill(i, _):
        z_v[pl.ds(i * 16, 16)] = jnp.zeros((16,), jnp.float32)
        return 0
    lax.fori_loop(0, RPT // 16, zfill, 0)

    def ofill(i, _):
        ones_v[pl.ds(i * 16, 16)] = jnp.ones((16,), jnp.float32)
        return 0
    lax.fori_loop(0, CD // 16, ofill, 0)

    pltpu.sync_copy(z_v, deg_sp.at[pl.ds(sid * RPT, RPT)])
    plsc.subcore_barrier()

    def drain_scatter(b):
        pltpu.make_async_copy(ones_v, deg_sp.at[didx[b]], sem_s[b]).wait()

    def run_chunks(base, nb, first):
        idescs = []
        for b in range(nb):
            if not first:
                drain_scatter(b)
            idescs.append(
                pltpu.async_copy(dst_hbm.at[pl.ds(base + b * CD, CD)],
                                 didx[b], sem_i[b]))
        for b in range(nb):
            idescs[b].wait()
            pltpu.async_copy(ones_v, deg_sp.at[didx[b]], sem_s[b],
                             add=True)

    def group(g, _):
        @pl.when(g == 0)
        def _():
            run_chunks(ebase, ND, True)

        @pl.when(g > 0)
        def _():
            run_chunks(ebase + g * ND * CD, ND, False)
        return 0
    lax.fori_loop(0, DGRP, group, 0)
    for b in range(ND):
        drain_scatter(b)
    if DTAIL:
        run_chunks(ebase + DGRP * ND * CD, DTAIL, True)
        for b in range(DTAIL):
            drain_scatter(b)
    if DTAILE:
        pltpu.sync_copy(dst_hbm.at[pl.ds(ebase + DCHUNKS * CD, DTAILE)],
                        didxt)
        pltpu.sync_copy(ones_v.at[pl.ds(0, DTAILE)], deg_sp.at[didxt],
                        add=True)

    plsc.subcore_barrier()
    pltpu.sync_copy(deg_sp.at[pl.ds(sid * RPT, RPT)], z_v)

    @pl.when(cid == 0)
    def _():
        pltpu.sync_copy(z_v, out_a.at[pl.ds(sid * RPT, RPT)])

    @pl.when(cid == 1)
    def _():
        pltpu.sync_copy(z_v, out_b.at[pl.ds(sid * RPT, RPT)])


@functools.cache
def _sc_deg():
    mesh = plsc.VectorSubcoreMesh(
        core_axis_name="c", subcore_axis_name="s",
        num_cores=NC, num_subcores=NS)
    return pl.kernel(
        _sc_deg_body,
        out_type=(
            jax.ShapeDtypeStruct((NP,), jnp.float32),
            jax.ShapeDtypeStruct((NP,), jnp.float32),
        ),
        mesh=mesh,
        scratch_types=(
            [pltpu.VMEM_SHARED((NP,), jnp.float32),
             pltpu.VMEM((RPT,), jnp.float32),
             pltpu.VMEM((CD,), jnp.float32),
             pltpu.VMEM((max(DTAILE, 8),), jnp.int32)]
            + [pltpu.VMEM((CD,), jnp.int32)] * ND
            + [pltpu.SemaphoreType.DMA] * (2 * ND)
        ),
    )


def _sc_agg_body(hs_hbm, src_hbm, dst_hbm, zer_hbm, out_hbm, acc_sp,
                 srcall, didxt, *bufs):
    didx = bufs[0:NB]
    rows = bufs[NB:2 * NB]
    sem_i = bufs[2 * NB:3 * NB]
    sem_r = bufs[3 * NB:4 * NB]
    sem_s = bufs[4 * NB:5 * NB]
    cid = lax.axis_index("c")
    sid = lax.axis_index("s")
    gw = cid * NS + sid
    ebase = gw * EPW

    pltpu.sync_copy(src_hbm.at[pl.ds(ebase, EPW)], srcall)

    pltpu.sync_copy(zer_hbm.at[pl.ds(sid * RPT, RPT)],
                    acc_sp.at[pl.ds(sid * RPT, RPT)])
    plsc.subcore_barrier()

    def drain_scatter(b):
        pltpu.make_async_copy(rows[b], acc_sp.at[didx[b]], sem_s[b]).wait()

    def run_chunks(off, base, nb, first):
        gdescs = []
        idescs = []
        for b in range(nb):
            if not first:
                drain_scatter(b)
            gdescs.append(
                pltpu.async_copy(
                    hs_hbm.at[srcall.at[pl.ds(off + b * C, C)]],
                    rows[b], sem_r[b]))
            idescs.append(
                pltpu.async_copy(dst_hbm.at[pl.ds(base + b * C, C)],
                                 didx[b], sem_i[b]))
        for b in range(nb):
            idescs[b].wait()
            gdescs[b].wait()
            pltpu.async_copy(rows[b], acc_sp.at[didx[b]], sem_s[b],
                             add=True)

    def group(g, _):
        @pl.when(g == 0)
        def _():
            run_chunks(0, ebase, NB, True)

        @pl.when(g > 0)
        def _():
            run_chunks(g * NB * C, ebase + g * NB * C, NB, False)
        return 0
    lax.fori_loop(0, GRP, group, 0)
    for b in range(NB):
        drain_scatter(b)
    toff = GRP * NB * C
    for t in range(TAIL):
        pltpu.sync_copy(dst_hbm.at[pl.ds(ebase + toff + t * C, C)], didx[t])
        pltpu.async_copy(hs_hbm.at[srcall.at[pl.ds(toff + t * C, C)]],
                         rows[t], sem_r[t]).wait()
        pltpu.sync_copy(rows[t], acc_sp.at[didx[t]], add=True)
    if TAILE:
        eoff = CHUNKS * C
        pltpu.sync_copy(dst_hbm.at[pl.ds(ebase + eoff, TAILE)], didxt)
        pltpu.async_copy(hs_hbm.at[srcall.at[pl.ds(eoff, TAILE)]],
                         rows[0].at[pl.ds(0, TAILE)], sem_r[0]).wait()
        pltpu.sync_copy(rows[0].at[pl.ds(0, TAILE)], acc_sp.at[didxt],
                        add=True)

    plsc.subcore_barrier()
    pltpu.sync_copy(acc_sp.at[pl.ds(sid * RPT, RPT)],
                    out_hbm.at[cid, pl.ds(sid * RPT, RPT)])


@functools.cache
def _sc_agg():
    mesh = plsc.VectorSubcoreMesh(
        core_axis_name="c", subcore_axis_name="s",
        num_cores=NC, num_subcores=NS)
    return pl.kernel(
        _sc_agg_body,
        out_type=jax.ShapeDtypeStruct((NC, NP, D), jnp.float32),
        mesh=mesh,
        scratch_types=(
            [pltpu.VMEM_SHARED((NP, D), jnp.float32),
             pltpu.VMEM((EPW,), jnp.int32),
             pltpu.VMEM((max(TAILE, 8),), jnp.int32)]
            + [pltpu.VMEM((C,), jnp.int32)] * NB
            + [pltpu.VMEM((C, D), jnp.float32)] * NB
            + [pltpu.SemaphoreType.DMA] * (3 * NB)
        ),
    )


R = 2560
G = NP // R


def _tc_emb_body(x_ref, we_ref, be_ref, h0_ref):
    h0_ref[...] = jnp.dot(x_ref[...], we_ref[...],
                          preferred_element_type=jnp.float32) + be_ref[...]


_tc_emb = pl.pallas_call(
    _tc_emb_body,
    grid=(G,),
    in_specs=[
        pl.BlockSpec((R, D), lambda i: (i, 0)),
        pl.BlockSpec((D, D), lambda i: (0, 0)),
        pl.BlockSpec((1, D), lambda i: (0, 0)),
    ],
    out_specs=pl.BlockSpec((R, D), lambda i: (i, 0)),
    out_shape=jax.ShapeDtypeStruct((NP, D), jnp.float32),
)


def _tc_pre_body(h0_ref, w1_ref, da_ref, db_ref, hs_ref, dinv_ref):
    dinv = lax.rsqrt(da_ref[...] + db_ref[...] + 1.0)
    hs_ref[...] = jnp.dot(h0_ref[...], w1_ref[...],
                          preferred_element_type=jnp.float32) * dinv
    dinv_ref[...] = dinv


_tc_pre = pl.pallas_call(
    _tc_pre_body,
    grid=(G,),
    in_specs=[
        pl.BlockSpec((R, D), lambda i: (i, 0)),
        pl.BlockSpec((D, D), lambda i: (0, 0)),
        pl.BlockSpec((R, 1), lambda i: (i, 0)),
        pl.BlockSpec((R, 1), lambda i: (i, 0)),
    ],
    out_specs=[
        pl.BlockSpec((R, D), lambda i: (i, 0)),
        pl.BlockSpec((R, 1), lambda i: (i, 0)),
    ],
    out_shape=[
        jax.ShapeDtypeStruct((NP, D), jnp.float32),
        jax.ShapeDtypeStruct((NP, 1), jnp.float32),
    ],
)


def _tc_mid_body(agg_ref, hs_ref, dinv_ref, b_ref, w_ref, out_ref):
    a = agg_ref[...]
    dinv = dinv_ref[...]
    h = (a[0] + a[1] + hs_ref[...]) * dinv + b_ref[...]
    out_ref[...] = jnp.dot(h, w_ref[...],
                           preferred_element_type=jnp.float32) * dinv


_tc_mid = pl.pallas_call(
    _tc_mid_body,
    grid=(G,),
    in_specs=[
        pl.BlockSpec((NC, R, D), lambda i: (0, i, 0)),
        pl.BlockSpec((R, D), lambda i: (i, 0)),
        pl.BlockSpec((R, 1), lambda i: (i, 0)),
        pl.BlockSpec((1, D), lambda i: (0, 0)),
        pl.BlockSpec((D, D), lambda i: (0, 0)),
    ],
    out_specs=pl.BlockSpec((R, D), lambda i: (i, 0)),
    out_shape=jax.ShapeDtypeStruct((NP, D), jnp.float32),
)


def _tc_dec_body(agg_ref, hs_ref, dinv_ref, b_ref, wd_ref, bd_ref, out_ref):
    a = agg_ref[...]
    h = (a[0] + a[1] + hs_ref[...]) * dinv_ref[...] + b_ref[...]
    out_ref[...] = jnp.dot(h, wd_ref[...],
                           preferred_element_type=jnp.float32) + bd_ref[...]


_tc_dec = pl.pallas_call(
    _tc_dec_body,
    grid=(G,),
    in_specs=[
        pl.BlockSpec((NC, R, D), lambda i: (0, i, 0)),
        pl.BlockSpec((R, D), lambda i: (i, 0)),
        pl.BlockSpec((R, 1), lambda i: (i, 0)),
        pl.BlockSpec((1, D), lambda i: (0, 0)),
        pl.BlockSpec((D, DO), lambda i: (0, 0)),
        pl.BlockSpec((1, DO), lambda i: (0, 0)),
    ],
    out_specs=pl.BlockSpec((R, DO), lambda i: (i, 0)),
    out_shape=jax.ShapeDtypeStruct((NP, DO), jnp.float32),
)


def kernel(x, edge_index, w_emb, b_emb, w1, b1, w2, b2, w3, b3, w_dec, b_dec):
    xp = jnp.pad(x, ((0, NP - N), (0, 0)))
    src = edge_index[0]
    dst = edge_index[1]
    zer = jnp.zeros((NP, D), jnp.float32)

    deg_a, deg_b = _sc_deg()(dst)
    h0 = _tc_emb(xp, w_emb, b_emb.reshape(1, D))
    da = deg_a.reshape(NP, 1)
    db = deg_b.reshape(NP, 1)

    hs, dinv = _tc_pre(h0, w1, da, db)
    agg = _sc_agg()(hs, src, dst, zer)
    hs = _tc_mid(agg, hs, dinv, b1.reshape(1, D), w2)
    agg = _sc_agg()(hs, src, dst, zer)
    hs = _tc_mid(agg, hs, dinv, b2.reshape(1, D), w3)
    agg = _sc_agg()(hs, src, dst, zer)
    out = _tc_dec(agg, hs, dinv, b3.reshape(1, D), w_dec, b_dec.reshape(1, DO))
    return out[:N]
